# baseline (speedup 1.0000x reference)
import functools
import math

import jax
import jax.numpy as jnp
import numpy as np
from jax import lax
from jax.experimental import pallas as pl
from jax.experimental.pallas import tpu as pltpu
from jax.experimental.pallas import tpu_sc as plsc

N_CH = 256
IN_BS = 49152
OUT_SR_BS = math.ceil(IN_BS * 44100 / 48000)

NW = 32
LANES = 16
J_PER_W = 1424
J_PAD = NW * J_PER_W
IN_STRIDE = 1544
WIN = 1744
START_CAP = IN_BS - WIN
N_VEC = J_PER_W // LANES


def _host_tables():
    sf = np.float32((IN_BS - 1) / (OUT_SR_BS - 1) + 1e-12)
    j = np.arange(OUT_SR_BS, dtype=np.float32)
    x = j * sf
    y0 = np.floor(x).astype(np.int32)
    y1 = np.clip(y0 + 1, 0, IN_BS - 1)
    frac = np.clip(x - y0.astype(np.float32), np.float32(0.0), np.float32(1.0))
    frac[0] = np.float32(0.0)
    frac[-1] = np.round(frac[-1])
    ym1 = np.clip(y0 - 1, 0, IN_BS - 1)
    y2 = np.clip(y1 + 1, 0, IN_BS - 1)

    def pad_i(a):
        return np.concatenate(
            [a, np.full(J_PAD - OUT_SR_BS, IN_BS - 1, np.int32)])

    frac_p = np.concatenate([frac, np.zeros(J_PAD - OUT_SR_BS, np.float32)])
    ym1_p, y0_p, y1_p, y2_p = pad_i(ym1), pad_i(y0), pad_i(y1), pad_i(y2)

    starts = np.minimum(IN_STRIDE * np.arange(NW), START_CAP).astype(np.int64)
    start_of_j = np.repeat(starts, J_PER_W)
    loc = []
    for a in (ym1_p, y0_p, y1_p, y2_p):
        la = a.astype(np.int64) - start_of_j
        assert la.min() >= 0 and la.max() < WIN, (la.min(), la.max())
        loc.append(la.astype(np.int32))
    return frac_p, loc[0], loc[1], loc[2], loc[3]


_XW, _IM1, _I0, _I1, _I2 = _host_tables()


def _resample_body(y_hbm, xw_hbm, im1_hbm, i0_hbm, i1_hbm, i2_hbm, out_hbm,
                   in_v, out_v, xw_v, im1_v, i0_v, i1_v, i2_v):
    wid = lax.axis_index("s") * 2 + lax.axis_index("c")
    j0 = wid * J_PER_W
    start = jnp.minimum(wid * IN_STRIDE, START_CAP)

    pltpu.sync_copy(xw_hbm.at[pl.ds(j0, J_PER_W)], xw_v)
    pltpu.sync_copy(im1_hbm.at[pl.ds(j0, J_PER_W)], im1_v)
    pltpu.sync_copy(i0_hbm.at[pl.ds(j0, J_PER_W)], i0_v)
    pltpu.sync_copy(i1_hbm.at[pl.ds(j0, J_PER_W)], i1_v)
    pltpu.sync_copy(i2_hbm.at[pl.ds(j0, J_PER_W)], i2_v)

    def chan_body(c, _):
        pltpu.sync_copy(y_hbm.at[pl.ds(c * IN_BS + start, WIN)], in_v)

        def vec_body(v, _):
            sl = pl.ds(v * LANES, LANES)
            ym1 = plsc.load_gather(in_v, [im1_v[sl]])
            y0 = plsc.load_gather(in_v, [i0_v[sl]])
            y1 = plsc.load_gather(in_v, [i1_v[sl]])
            y2 = plsc.load_gather(in_v, [i2_v[sl]])
            x = xw_v[sl]
            c1 = 0.5 * (y1 - ym1)
            c2 = ym1 - 2.5 * y0 + 2.0 * y1 - 0.5 * y2
            c3 = 1.5 * (y0 - y1) + 0.5 * (y2 - ym1)
            out_v[sl] = ((c3 * x + c2) * x + c1) * x + y0
            return 0

        lax.fori_loop(0, N_VEC, vec_body, 0)
        pltpu.sync_copy(out_v, out_hbm.at[pl.ds(c * J_PAD + j0, J_PER_W)])
        return 0

    lax.fori_loop(0, N_CH, chan_body, 0)


@functools.lru_cache(maxsize=1)
def _build():
    mesh = plsc.VectorSubcoreMesh(
        core_axis_name="c", subcore_axis_name="s",
        num_cores=2, num_subcores=16)
    return pl.kernel(
        _resample_body,
        out_type=jax.ShapeDtypeStruct((N_CH * J_PAD,), jnp.float32),
        mesh=mesh,
        compiler_params=pltpu.CompilerParams(needs_layout_passes=False),
        scratch_types=[
            pltpu.VMEM((WIN,), jnp.float32),
            pltpu.VMEM((J_PER_W,), jnp.float32),
            pltpu.VMEM((J_PER_W,), jnp.float32),
            pltpu.VMEM((J_PER_W,), jnp.int32),
            pltpu.VMEM((J_PER_W,), jnp.int32),
            pltpu.VMEM((J_PER_W,), jnp.int32),
            pltpu.VMEM((J_PER_W,), jnp.int32),
        ],
    )


def kernel(y):
    xw = jnp.asarray(_XW)
    im1 = jnp.asarray(_IM1)
    i0 = jnp.asarray(_I0)
    i1 = jnp.asarray(_I1)
    i2 = jnp.asarray(_I2)
    out = _build()(y.reshape(-1), xw, im1, i0, i1, i2)
    return out.reshape(N_CH, J_PAD)[:, :OUT_SR_BS]


# speedup vs baseline: 1.0655x; 1.0655x over previous
import functools
import math

import jax
import jax.numpy as jnp
import numpy as np
from jax import lax
from jax.experimental import pallas as pl
from jax.experimental.pallas import tpu as pltpu
from jax.experimental.pallas import tpu_sc as plsc

N_CH = 256
IN_BS = 49152
OUT_BS = math.ceil(IN_BS * 44100 / 48000)

NW = 32
LANES = 16
JW = 1408
JBUF = 1536
J_PAD = NW * JW + (JBUF - JW)
NVEC_STD = JW // LANES
NVEC_LAST = JBUF // LANES
WIN = 1664
CB = 8
NCB = N_CH // CB


def _start(w):
    return 128 * ((49039 * w) >> 12)


def _host_tables():
    sf = np.float32((IN_BS - 1) / (OUT_BS - 1) + 1e-12)
    jg = np.arange(OUT_BS, dtype=np.float32)
    xg = jg * sf
    y0g = np.floor(xg).astype(np.int64)
    y1g = np.clip(y0g + 1, 0, IN_BS - 1)
    fr = np.clip(xg - y0g.astype(np.float32), np.float32(0.0), np.float32(1.0))
    fr[0] = np.float32(0.0)
    fr[-1] = np.round(fr[-1])
    ym1g = np.clip(y0g - 1, 0, IN_BS - 1)
    y2g = np.clip(y1g + 1, 0, IN_BS - 1)

    i0 = np.zeros(NW * JBUF, np.int32)
    xw = np.zeros(NW * JBUF, np.float32)
    for w in range(NW):
        s = _start(w)
        assert s >= 0 and s + WIN <= IN_BS
        nj = (NVEC_LAST if w == NW - 1 else NVEC_STD) * LANES
        for_jj = np.arange(nj)
        j = w * JW + for_jj
        real = j < OUT_BS
        jr = j[real]
        loc0 = y0g[jr] - s
        assert loc0.min() >= (1 if w > 0 else 0) and loc0.max() <= WIN - 1
        assert np.array_equal(np.maximum(loc0 - 1, 0) + s, ym1g[jr])
        assert np.array_equal(np.minimum(loc0 + 1, WIN - 1) + s, y1g[jr])
        assert np.array_equal(np.minimum(loc0 + 2, WIN - 1) + s, y2g[jr])
        blk0 = np.full(nj, 1, np.int32)
        blk0[real] = loc0
        blkx = np.zeros(nj, np.float32)
        blkx[real] = fr[jr]
        i0[w * JBUF: w * JBUF + nj] = blk0
        xw[w * JBUF: w * JBUF + nj] = blkx
    return xw, i0


_XW, _I0 = _host_tables()


def _wait_out(ob, out_hbm, os_, wid):
    @pl.when(wid == NW - 1)
    def _():
        pltpu.make_async_copy(
            ob.at[pl.ds(0, CB), pl.ds(0, JBUF)],
            out_hbm.at[pl.ds(0, CB), pl.ds(0, JBUF)], os_).wait()

    @pl.when(wid != NW - 1)
    def _():
        pltpu.make_async_copy(
            ob.at[pl.ds(0, CB), pl.ds(0, JW)],
            out_hbm.at[pl.ds(0, CB), pl.ds(0, JW)], os_).wait()


def _resample_body(y_hbm, xw_hbm, i0_hbm, out_hbm,
                   in0, in1, ob0, ob1, i0_v, xw_v, is0, is1, os0, os1):
    wid = lax.axis_index("s") * 2 + lax.axis_index("c")
    j0 = pl.multiple_of(wid * JW, 128)
    tb = pl.multiple_of(wid * JBUF, 128)
    s_w = pl.multiple_of(128 * ((wid * 49039) >> 12), 128)
    n_vec = jnp.where(wid == NW - 1, NVEC_LAST, NVEC_STD)

    pltpu.sync_copy(i0_hbm.at[pl.ds(tb, JBUF)], i0_v)
    pltpu.sync_copy(xw_hbm.at[pl.ds(tb, JBUF)], xw_v)

    pltpu.async_copy(y_hbm.at[pl.ds(0, CB), pl.ds(s_w, WIN)], in0, is0)
    pltpu.async_copy(y_hbm.at[pl.ds(CB, CB), pl.ds(s_w, WIN)], in1, is1)

    def compute(in_b, ob):
        for ch in range(CB):
            rowv = jnp.full((LANES,), ch, jnp.int32)

            def vbody(v, _):
                sl = pl.ds(v * LANES, LANES)
                i0 = i0_v[sl]
                x = xw_v[sl]
                im1 = jnp.maximum(i0 - 1, 0)
                i1 = jnp.minimum(i0 + 1, WIN - 1)
                i2 = jnp.minimum(i0 + 2, WIN - 1)
                ym1 = plsc.load_gather(in_b, [rowv, im1])
                y0 = plsc.load_gather(in_b, [rowv, i0])
                y1 = plsc.load_gather(in_b, [rowv, i1])
                y2 = plsc.load_gather(in_b, [rowv, i2])
                c1 = 0.5 * (y1 - ym1)
                c2 = ym1 - 2.5 * y0 + 2.0 * y1 - 0.5 * y2
                c3 = 1.5 * (y0 - y1) + 0.5 * (y2 - ym1)
                ob[ch, sl] = ((c3 * x + c2) * x + c1) * x + y0
                return 0

            lax.fori_loop(0, n_vec, vbody, 0)

    def phase(t, cb, in_b, ob, is_, os_):
        c8 = pl.multiple_of(cb * CB, 8)
        pltpu.make_async_copy(
            y_hbm.at[pl.ds(0, CB), pl.ds(0, WIN)], in_b, is_).wait()

        @pl.when(t > 0)
        def _():
            _wait_out(ob, out_hbm, os_, wid)

        compute(in_b, ob)

        @pl.when(wid == NW - 1)
        def _():
            pltpu.async_copy(
                ob.at[pl.ds(0, CB), pl.ds(0, JBUF)],
                out_hbm.at[pl.ds(c8, CB), pl.ds(j0, JBUF)], os_)

        @pl.when(wid != NW - 1)
        def _():
            pltpu.async_copy(
                ob.at[pl.ds(0, CB), pl.ds(0, JW)],
                out_hbm.at[pl.ds(c8, CB), pl.ds(j0, JW)], os_)

        @pl.when(cb + 2 < NCB)
        def _():
            nxt = pl.multiple_of((cb + 2) * CB, 8)
            pltpu.async_copy(
                y_hbm.at[pl.ds(nxt, CB), pl.ds(s_w, WIN)], in_b, is_)

    def tbody(t, _):
        phase(t, 2 * t, in0, ob0, is0, os0)
        phase(t, 2 * t + 1, in1, ob1, is1, os1)
        return 0

    lax.fori_loop(0, NCB // 2, tbody, 0)
    _wait_out(ob0, out_hbm, os0, wid)
    _wait_out(ob1, out_hbm, os1, wid)


@functools.lru_cache(maxsize=1)
def _build():
    mesh = plsc.VectorSubcoreMesh(
        core_axis_name="c", subcore_axis_name="s",
        num_cores=2, num_subcores=16)
    return pl.kernel(
        _resample_body,
        out_type=jax.ShapeDtypeStruct((N_CH, J_PAD), jnp.float32),
        mesh=mesh,
        compiler_params=pltpu.CompilerParams(needs_layout_passes=False),
        scratch_types=[
            pltpu.VMEM((CB, WIN), jnp.float32),
            pltpu.VMEM((CB, WIN), jnp.float32),
            pltpu.VMEM((CB, JBUF), jnp.float32),
            pltpu.VMEM((CB, JBUF), jnp.float32),
            pltpu.VMEM((JBUF,), jnp.int32),
            pltpu.VMEM((JBUF,), jnp.float32),
            pltpu.SemaphoreType.DMA,
            pltpu.SemaphoreType.DMA,
            pltpu.SemaphoreType.DMA,
            pltpu.SemaphoreType.DMA,
        ],
    )


def kernel(y):
    out = _build()(y, jnp.asarray(_XW), jnp.asarray(_I0))
    return out[:, :OUT_BS]


# speedup vs baseline: 2.3275x; 2.1844x over previous
import functools
import math

import jax
import jax.numpy as jnp
import numpy as np
from jax import lax
from jax.experimental import pallas as pl
from jax.experimental.pallas import tpu as pltpu
from jax.experimental.pallas import tpu_sc as plsc

N_CH = 256
IN_BS = 49152
OUT_BS = math.ceil(IN_BS * 44100 / 48000)

NW = 32
LANES = 16
JW = 1408
JBUF = 1536
J_PAD = NW * JW + (JBUF - JW)
NVEC_STD = JW // LANES
NVEC_LAST = JBUF // LANES
WIN = 1664
CB = 8
NCB = N_CH // CB


def _start(w):
    return 128 * ((49039 * w) >> 12)


def _host_tables():
    sf = np.float32((IN_BS - 1) / (OUT_BS - 1) + 1e-12)
    jg = np.arange(OUT_BS, dtype=np.float32)
    xg = jg * sf
    y0g = np.floor(xg).astype(np.int64)
    y1g = np.clip(y0g + 1, 0, IN_BS - 1)
    fr = np.clip(xg - y0g.astype(np.float32), np.float32(0.0), np.float32(1.0))
    fr[0] = np.float32(0.0)
    fr[-1] = np.round(fr[-1])
    ym1g = np.clip(y0g - 1, 0, IN_BS - 1)
    y2g = np.clip(y1g + 1, 0, IN_BS - 1)

    i0 = np.zeros(NW * JBUF, np.int32)
    xw = np.zeros(NW * JBUF, np.float32)
    for w in range(NW):
        s = _start(w)
        assert s >= 0 and s + WIN <= IN_BS
        nj = (NVEC_LAST if w == NW - 1 else NVEC_STD) * LANES
        for_jj = np.arange(nj)
        j = w * JW + for_jj
        real = j < OUT_BS
        jr = j[real]
        loc0 = y0g[jr] - s
        assert loc0.min() >= (1 if w > 0 else 0) and loc0.max() <= WIN - 1
        assert np.array_equal(np.maximum(loc0 - 1, 0) + s, ym1g[jr])
        assert np.array_equal(np.minimum(loc0 + 1, WIN - 1) + s, y1g[jr])
        assert np.array_equal(np.minimum(loc0 + 2, WIN - 1) + s, y2g[jr])
        blk0 = np.full(nj, 1, np.int32)
        blk0[real] = loc0
        blkx = np.zeros(nj, np.float32)
        blkx[real] = fr[jr]
        i0[w * JBUF: w * JBUF + nj] = blk0
        xw[w * JBUF: w * JBUF + nj] = blkx
    return xw, i0


_XW, _I0 = _host_tables()


def _wait_out(ob, out_hbm, os_, wid):
    @pl.when(wid == NW - 1)
    def _():
        pltpu.make_async_copy(
            ob.at[pl.ds(0, CB), pl.ds(0, JBUF)],
            out_hbm.at[pl.ds(0, CB), pl.ds(0, JBUF)], os_).wait()

    @pl.when(wid != NW - 1)
    def _():
        pltpu.make_async_copy(
            ob.at[pl.ds(0, CB), pl.ds(0, JW)],
            out_hbm.at[pl.ds(0, CB), pl.ds(0, JW)], os_).wait()


def _resample_body(y_hbm, xw_hbm, i0_hbm, out_hbm,
                   in0, in1, ob0, ob1, i0_v, xw_v, is0, is1, os0, os1):
    wid = lax.axis_index("s") * 2 + lax.axis_index("c")
    j0 = pl.multiple_of(wid * JW, 128)
    tb = pl.multiple_of(wid * JBUF, 128)
    s_w = pl.multiple_of(128 * ((wid * 49039) >> 12), 128)
    n_vec = jnp.where(wid == NW - 1, NVEC_LAST, NVEC_STD)

    pltpu.sync_copy(i0_hbm.at[pl.ds(tb, JBUF)], i0_v)
    pltpu.sync_copy(xw_hbm.at[pl.ds(tb, JBUF)], xw_v)

    pltpu.async_copy(y_hbm.at[pl.ds(0, CB), pl.ds(s_w, WIN)], in0, is0)
    pltpu.async_copy(y_hbm.at[pl.ds(CB, CB), pl.ds(s_w, WIN)], in1, is1)

    def compute(in_b, ob):
        for ch in range(CB):
            rowv = jnp.full((LANES,), ch, jnp.int32)

            @plsc.parallel_loop(0, n_vec * LANES, LANES, unroll=4)
            def _(j):
                sl = pl.ds(j, LANES)
                i0 = i0_v[sl]
                x = xw_v[sl]
                im1 = jnp.maximum(i0 - 1, 0)
                i1 = jnp.minimum(i0 + 1, WIN - 1)
                i2 = jnp.minimum(i0 + 2, WIN - 1)
                ym1 = plsc.load_gather(in_b, [rowv, im1])
                y0 = plsc.load_gather(in_b, [rowv, i0])
                y1 = plsc.load_gather(in_b, [rowv, i1])
                y2 = plsc.load_gather(in_b, [rowv, i2])
                c1 = 0.5 * (y1 - ym1)
                c2 = ym1 - 2.5 * y0 + 2.0 * y1 - 0.5 * y2
                c3 = 1.5 * (y0 - y1) + 0.5 * (y2 - ym1)
                ob[ch, sl] = ((c3 * x + c2) * x + c1) * x + y0

    def phase(t, cb, in_b, ob, is_, os_):
        c8 = pl.multiple_of(cb * CB, 8)
        pltpu.make_async_copy(
            y_hbm.at[pl.ds(0, CB), pl.ds(0, WIN)], in_b, is_).wait()

        @pl.when(t > 0)
        def _():
            _wait_out(ob, out_hbm, os_, wid)

        compute(in_b, ob)

        @pl.when(wid == NW - 1)
        def _():
            pltpu.async_copy(
                ob.at[pl.ds(0, CB), pl.ds(0, JBUF)],
                out_hbm.at[pl.ds(c8, CB), pl.ds(j0, JBUF)], os_)

        @pl.when(wid != NW - 1)
        def _():
            pltpu.async_copy(
                ob.at[pl.ds(0, CB), pl.ds(0, JW)],
                out_hbm.at[pl.ds(c8, CB), pl.ds(j0, JW)], os_)

        @pl.when(cb + 2 < NCB)
        def _():
            nxt = pl.multiple_of((cb + 2) * CB, 8)
            pltpu.async_copy(
                y_hbm.at[pl.ds(nxt, CB), pl.ds(s_w, WIN)], in_b, is_)

    def tbody(t, _):
        phase(t, 2 * t, in0, ob0, is0, os0)
        phase(t, 2 * t + 1, in1, ob1, is1, os1)
        return 0

    lax.fori_loop(0, NCB // 2, tbody, 0)
    _wait_out(ob0, out_hbm, os0, wid)
    _wait_out(ob1, out_hbm, os1, wid)


@functools.lru_cache(maxsize=1)
def _build():
    mesh = plsc.VectorSubcoreMesh(
        core_axis_name="c", subcore_axis_name="s",
        num_cores=2, num_subcores=16)
    return pl.kernel(
        _resample_body,
        out_type=jax.ShapeDtypeStruct((N_CH, J_PAD), jnp.float32),
        mesh=mesh,
        compiler_params=pltpu.CompilerParams(needs_layout_passes=False),
        scratch_types=[
            pltpu.VMEM((CB, WIN), jnp.float32),
            pltpu.VMEM((CB, WIN), jnp.float32),
            pltpu.VMEM((CB, JBUF), jnp.float32),
            pltpu.VMEM((CB, JBUF), jnp.float32),
            pltpu.VMEM((JBUF,), jnp.int32),
            pltpu.VMEM((JBUF,), jnp.float32),
            pltpu.SemaphoreType.DMA,
            pltpu.SemaphoreType.DMA,
            pltpu.SemaphoreType.DMA,
            pltpu.SemaphoreType.DMA,
        ],
    )


def kernel(y):
    out = _build()(y, jnp.asarray(_XW), jnp.asarray(_I0))
    return out[:, :OUT_BS]
